# baseline (speedup 1.0000x reference)
import jax
import jax.numpy as jnp
from jax.experimental import pallas as pl


_BLOCK_ROWS = 2048


def _body(x_ref, hp_ref, sel_ref, out_ref):
    x = x_ref[...]
    k = hp_ref[0, :]
    m = hp_ref[1, :]
    km = k * m
    c = 0.5 * jnp.sum(km * m) * 0.5
    t = x * (0.5 * k[None, :] * x - km[None, :])
    out_ref[...] = (
        jax.lax.dot_general(
            t, sel_ref[...], (((1,), (0,)), ((), ())),
            preferred_element_type=jnp.float32,
        )
        + c
    )


def kernel(in_feat, harmonic_parameters):
    n, f = in_feat.shape
    n2, f2 = n // 2, f * 2
    xr = in_feat.reshape(n2, f2)
    hp2 = jnp.tile(harmonic_parameters, (1, 2))
    half = (jax.lax.iota(jnp.int32, f2) >= f).astype(jnp.float32)
    sel = jnp.stack([1.0 - half, half], axis=1)
    out = pl.pallas_call(
        _body,
        grid=(n2 // _BLOCK_ROWS,),
        in_specs=[
            pl.BlockSpec((_BLOCK_ROWS, f2), lambda i: (i, 0)),
            pl.BlockSpec((2, f2), lambda i: (0, 0)),
            pl.BlockSpec((f2, 2), lambda i: (0, 0)),
        ],
        out_specs=pl.BlockSpec((_BLOCK_ROWS, 2), lambda i: (i, 0)),
        out_shape=jax.ShapeDtypeStruct((n2, 2), jnp.float32),
    )(xr, hp2, sel)
    return out.reshape(n, 1)


# speedup vs baseline: 1.3109x; 1.3109x over previous
import jax
import jax.numpy as jnp
from jax.experimental import pallas as pl


_BLOCK_ROWS = 16384


def _body(x_ref, hp_ref, out_ref):
    x = x_ref[...]
    k = hp_ref[0, :]
    m = hp_ref[1, :]
    km = k * m
    c = 0.5 * jnp.sum(km * m)
    t = x * (0.5 * k[None, :] * x - km[None, :])
    ones = jnp.ones((x.shape[1], 1), dtype=jnp.float32)
    out_ref[...] = (
        jax.lax.dot_general(
            t, ones, (((1,), (0,)), ((), ())), preferred_element_type=jnp.float32
        )
        + c
    )


def kernel(in_feat, harmonic_parameters):
    n, f = in_feat.shape
    grid = (n // _BLOCK_ROWS,)
    return pl.pallas_call(
        _body,
        grid=grid,
        in_specs=[
            pl.BlockSpec((_BLOCK_ROWS, f), lambda i: (i, 0)),
            pl.BlockSpec((2, f), lambda i: (0, 0)),
        ],
        out_specs=pl.BlockSpec((_BLOCK_ROWS, 1), lambda i: (i, 0)),
        out_shape=jax.ShapeDtypeStruct((n, 1), jnp.float32),
    )(in_feat, harmonic_parameters)


# speedup vs baseline: 1.4540x; 1.1092x over previous
import jax
import jax.numpy as jnp
from jax.experimental import pallas as pl


_BLOCK_ROWS = 16384


def _body(x_ref, hp_ref, out_ref):
    x = x_ref[...]
    k = hp_ref[0, :]
    m = hp_ref[1, :]
    km = k * m
    c = 0.5 * jnp.sum(km * m)
    t = x * (0.5 * k[None, :] * x - km[None, :])
    out_ref[...] = jnp.sum(t, axis=1) + c


def kernel(in_feat, harmonic_parameters):
    n, f = in_feat.shape
    grid = (n // _BLOCK_ROWS,)
    out = pl.pallas_call(
        _body,
        grid=grid,
        in_specs=[
            pl.BlockSpec((_BLOCK_ROWS, f), lambda i: (i, 0)),
            pl.BlockSpec((2, f), lambda i: (0, 0)),
        ],
        out_specs=pl.BlockSpec((_BLOCK_ROWS,), lambda i: (i,)),
        out_shape=jax.ShapeDtypeStruct((n,), jnp.float32),
    )(in_feat, harmonic_parameters)
    return out.reshape(n, 1)


# speedup vs baseline: 1.4910x; 1.0254x over previous
import jax
import jax.numpy as jnp
from jax.experimental import pallas as pl
from jax.experimental.pallas import tpu as pltpu


_NCHUNK = 16


def _body(x_hbm, hp_ref, out_ref, x_vmem, sems):
    n, f = x_hbm.shape
    rows = n // _NCHUNK

    def copy(c):
        return pltpu.make_async_copy(
            x_hbm.at[pl.ds(c * rows, rows), :],
            x_vmem.at[pl.ds(c * rows, rows), :],
            sems.at[c],
        )

    for c in range(_NCHUNK):
        copy(c).start()

    k = hp_ref[0, :]
    m = hp_ref[1, :]
    km = k * m
    const = 0.5 * jnp.sum(km * m)
    for c in range(_NCHUNK):
        copy(c).wait()
        x = x_vmem[pl.ds(c * rows, rows), :]
        t = x * (0.5 * k[None, :] * x - km[None, :])
        out_ref[pl.ds(c * rows, rows)] = jnp.sum(t, axis=1) + const


def kernel(in_feat, harmonic_parameters):
    n, f = in_feat.shape
    out = pl.pallas_call(
        _body,
        in_specs=[
            pl.BlockSpec(memory_space=pltpu.MemorySpace.HBM),
            pl.BlockSpec((2, f), lambda: (0, 0)),
        ],
        out_specs=pl.BlockSpec((n,), lambda: (0,)),
        out_shape=jax.ShapeDtypeStruct((n,), jnp.float32),
        scratch_shapes=[
            pltpu.VMEM((n, f), jnp.float32),
            pltpu.SemaphoreType.DMA((_NCHUNK,)),
        ],
        grid=(),
    )(in_feat, harmonic_parameters)
    return out.reshape(n, 1)
